# Initial kernel scaffold; baseline (speedup 1.0000x reference)
import jax
import jax.numpy as jnp
from jax.experimental import pallas as pl


def kernel(agent_h, team_idx, n_teams, W1, b1, W2, b2, W3, b3):
    raise NotImplementedError("write your pallas kernel here")


# speedup vs baseline: 2.3756x; 2.3756x over previous
import functools

import jax
import jax.numpy as jnp
from jax import lax
from jax.experimental import pallas as pl
from jax.experimental.pallas import tpu as pltpu
from jax.experimental.pallas import tpu_sc as plsc

N = 100000
H = 128
T = 10000
EXT = 144
BLK1 = 160
SBLK = 80
TBLK = 400
TPW = T // 16


def _scores_prescale_body(h_ref, w1_ref, b1_ref, w2_ref, b2_ref, sc_ref,
                          g_ref):
    h = h_ref[...]
    t1 = jnp.tanh(
        jax.lax.dot_general(h, w1_ref[...], (((1,), (0,)), ((), ())),
                            preferred_element_type=jnp.float32,
                            precision=jax.lax.Precision.HIGHEST)
        + b1_ref[...])
    w2 = w2_ref[...]
    scale = sc_ref[0, 0]
    s = (jnp.sum(t1 * w2, axis=1, keepdims=True) + b2_ref[...]) * scale
    m = (jnp.sum(jnp.abs(w2)) + jnp.abs(b2_ref[0, 0])) * scale
    ex = jnp.exp(s - m)
    g_ref[:, :H] = h * ex
    g_ref[:, H:] = jnp.broadcast_to(ex, (BLK1, EXT - H))


def _scores_prescale(agent_h, w1, b1r, w2r, b2r, scale2d):
    return pl.pallas_call(
        _scores_prescale_body,
        grid=(N // BLK1,),
        in_specs=[
            pl.BlockSpec((BLK1, H), lambda i: (i, 0)),
            pl.BlockSpec((H, H // 2), lambda i: (0, 0)),
            pl.BlockSpec((1, H // 2), lambda i: (0, 0)),
            pl.BlockSpec((1, H // 2), lambda i: (0, 0)),
            pl.BlockSpec((1, 1), lambda i: (0, 0)),
            pl.BlockSpec((1, 1), lambda i: (0, 0)),
        ],
        out_specs=pl.BlockSpec((BLK1, EXT), lambda i: (i, 0)),
        out_shape=jax.ShapeDtypeStruct((N, EXT), jnp.float32),
    )(agent_h, w1, b1r, w2r, b2r, scale2d)


def _segment_accumulate(g, idx2d, zeros_init):
    mesh = plsc.VectorSubcoreMesh(core_axis_name="c", subcore_axis_name="s")

    @functools.partial(
        pl.kernel,
        out_type=jax.ShapeDtypeStruct((2, T, EXT), jnp.float32),
        mesh=mesh,
        scratch_types=[pltpu.VMEM_SHARED((T, EXT), jnp.float32)],
        compiler_params=pltpu.CompilerParams(use_tc_tiling_on_sc=False),
    )
    def k(g_hbm, idx_hbm, z_hbm, out_hbm, u_acc):
        cid = lax.axis_index("c")
        sid = lax.axis_index("s")
        pltpu.sync_copy(z_hbm.at[pl.ds(sid * 624, 624)],
                        u_acc.at[pl.ds(sid * 624, 624)])

        @pl.when(sid == 15)
        def _():
            pltpu.sync_copy(z_hbm.at[pl.ds(9984, 16)],
                            u_acc.at[pl.ds(9984, 16)])

        plsc.subcore_barrier()

        def body(g_vmem, idx_vmem):
            pltpu.sync_copy(g_vmem, u_acc.at[idx_vmem.at[0]], add=True)

        pltpu.emit_pipeline(
            body,
            grid=(N // SBLK,),
            in_specs=[
                pl.BlockSpec((SBLK, EXT), index_map=lambda i: (i, 0)),
                pl.BlockSpec((1, SBLK), index_map=lambda i: (i, 0)),
            ],
            out_specs=[],
            core_axis_name=("c", "s"),
            dimension_semantics=(pltpu.PARALLEL,),
        )(g_hbm, idx_hbm)

        plsc.subcore_barrier()
        pltpu.sync_copy(u_acc.at[pl.ds(sid * 624, 624)],
                        out_hbm.at[cid, pl.ds(sid * 624, 624)])

        @pl.when(sid == 15)
        def _():
            pltpu.sync_copy(u_acc.at[pl.ds(9984, 16)],
                            out_hbm.at[cid, pl.ds(9984, 16)])

    return k(g, idx2d, zeros_init)


def _finalize_body(u_ref, w3_ref, b3_ref, th_ref, rd_ref):
    u = u_ref[0] + u_ref[1]
    d = u[:, H:H + 1]
    rd = jnp.where(d > 0.0, 1.0 / d, 0.0)
    uh = u[:, :H] * rd
    th = jax.lax.dot_general(uh, w3_ref[...], (((1,), (0,)), ((), ())),
                             preferred_element_type=jnp.float32,
                             precision=jax.lax.Precision.HIGHEST)
    th_ref[...] = jnp.maximum(th + b3_ref[...], 0.0)
    rd_ref[...] = rd


def _finalize(u2, w3, b3r):
    return pl.pallas_call(
        _finalize_body,
        grid=(T // TBLK,),
        in_specs=[
            pl.BlockSpec((2, TBLK, EXT), lambda i: (0, i, 0)),
            pl.BlockSpec((H, H), lambda i: (0, 0)),
            pl.BlockSpec((1, H), lambda i: (0, 0)),
        ],
        out_specs=[
            pl.BlockSpec((TBLK, H), lambda i: (i, 0)),
            pl.BlockSpec((TBLK, 1), lambda i: (i, 0)),
        ],
        out_shape=[
            jax.ShapeDtypeStruct((T, H), jnp.float32),
            jax.ShapeDtypeStruct((T, 1), jnp.float32),
        ],
    )(u2, w3, b3r)


def _attn_gather(g, idx2d, rdenom):
    mesh = plsc.VectorSubcoreMesh(core_axis_name="c", subcore_axis_name="s")

    @functools.partial(
        pl.kernel,
        out_type=jax.ShapeDtypeStruct((N // SBLK, SBLK), jnp.float32),
        mesh=mesh,
        scratch_types=[pltpu.VMEM((T,), jnp.float32)],
        compiler_params=pltpu.CompilerParams(use_tc_tiling_on_sc=False,
                                             needs_layout_passes=False),
    )
    def k(g_hbm, idx_hbm, rd_hbm, out_hbm, rd_vmem):
        pltpu.sync_copy(rd_hbm, rd_vmem)

        def body(ex_vmem, idx_vmem, attn_vmem):
            lanes = jnp.arange(16, dtype=jnp.int32)
            zeros = jnp.zeros((16,), jnp.int32)
            for j in range(SBLK // 16):
                idxv = idx_vmem[0, pl.ds(j * 16, 16)]
                rd = plsc.load_gather(rd_vmem, [idxv])
                exv = plsc.load_gather(ex_vmem, [j * 16 + lanes, zeros])
                attn_vmem[0, pl.ds(j * 16, 16)] = exv * rd

        pltpu.emit_pipeline(
            body,
            grid=(N // SBLK,),
            in_specs=[
                pl.BlockSpec((SBLK, EXT - H), index_map=lambda i: (i, H // (EXT - H))),
                pl.BlockSpec((1, SBLK), index_map=lambda i: (i, 0)),
            ],
            out_specs=[pl.BlockSpec((1, SBLK), index_map=lambda i: (i, 0))],
            core_axis_name=("c", "s"),
            dimension_semantics=(pltpu.PARALLEL,),
        )(g_hbm, idx_hbm, out_hbm)

    return k(g, idx2d, rdenom)


def kernel(agent_h, team_idx, n_teams, W1, b1, W2, b2, W3, b3):
    idx2d = team_idx.astype(jnp.int32).reshape(N // SBLK, SBLK)
    scale2d = (jnp.asarray(n_teams, jnp.float32) / 10000.0).reshape(1, 1)
    b1r = b1.reshape(1, H // 2)
    w2r = W2.reshape(1, H // 2)
    b2r = b2.reshape(1, 1)
    b3r = b3.reshape(1, H)

    g = _scores_prescale(agent_h, W1, b1r, w2r, b2r, scale2d)
    u2 = _segment_accumulate(g, idx2d, jnp.zeros((T, EXT), jnp.float32))
    team_h, rd2d = _finalize(u2, W3, b3r)
    attn2d = _attn_gather(g, idx2d, rd2d.reshape(T))
    return team_h, attn2d.reshape(N)


# speedup vs baseline: 3.8581x; 1.6240x over previous
import functools

import jax
import jax.numpy as jnp
from jax import lax
from jax.experimental import pallas as pl
from jax.experimental.pallas import tpu as pltpu
from jax.experimental.pallas import tpu_sc as plsc

N = 100000
H = 128
T = 10000
EXT = 144
BLK1 = 2000
SBLK = 80
TBLK = 400
TPW = T // 16


def _scores_prescale_body(h_ref, w1_ref, b1_ref, w2_ref, b2_ref, sc_ref,
                          g_ref):
    h = h_ref[...]
    t1 = jnp.tanh(
        jax.lax.dot_general(h, w1_ref[...], (((1,), (0,)), ((), ())),
                            preferred_element_type=jnp.float32,
                            precision=jax.lax.Precision.HIGHEST)
        + b1_ref[...])
    w2 = w2_ref[...]
    scale = sc_ref[0, 0]
    s = (jnp.sum(t1 * w2, axis=1, keepdims=True) + b2_ref[...]) * scale
    m = (jnp.sum(jnp.abs(w2)) + jnp.abs(b2_ref[0, 0])) * scale
    ex = jnp.exp(s - m)
    g_ref[:, :H] = h * ex
    g_ref[:, H:] = jnp.broadcast_to(ex, (BLK1, EXT - H))


def _scores_prescale(agent_h, w1, b1r, w2r, b2r, scale2d):
    return pl.pallas_call(
        _scores_prescale_body,
        grid=(N // BLK1,),
        in_specs=[
            pl.BlockSpec((BLK1, H), lambda i: (i, 0)),
            pl.BlockSpec((H, H // 2), lambda i: (0, 0)),
            pl.BlockSpec((1, H // 2), lambda i: (0, 0)),
            pl.BlockSpec((1, H // 2), lambda i: (0, 0)),
            pl.BlockSpec((1, 1), lambda i: (0, 0)),
            pl.BlockSpec((1, 1), lambda i: (0, 0)),
        ],
        out_specs=pl.BlockSpec((BLK1, EXT), lambda i: (i, 0)),
        out_shape=jax.ShapeDtypeStruct((N, EXT), jnp.float32),
    )(agent_h, w1, b1r, w2r, b2r, scale2d)


def _segment_accumulate(g, idx2d, zeros_init):
    mesh = plsc.VectorSubcoreMesh(core_axis_name="c", subcore_axis_name="s")

    @functools.partial(
        pl.kernel,
        out_type=jax.ShapeDtypeStruct((2, T, EXT), jnp.float32),
        mesh=mesh,
        scratch_types=[pltpu.VMEM_SHARED((T, EXT), jnp.float32)],
        compiler_params=pltpu.CompilerParams(use_tc_tiling_on_sc=False),
    )
    def k(g_hbm, idx_hbm, z_hbm, out_hbm, u_acc):
        cid = lax.axis_index("c")
        sid = lax.axis_index("s")
        pltpu.sync_copy(z_hbm.at[pl.ds(sid * 624, 624)],
                        u_acc.at[pl.ds(sid * 624, 624)])

        @pl.when(sid == 15)
        def _():
            pltpu.sync_copy(z_hbm.at[pl.ds(9984, 16)],
                            u_acc.at[pl.ds(9984, 16)])

        plsc.subcore_barrier()

        def body(g_vmem, idx_vmem):
            pltpu.sync_copy(g_vmem, u_acc.at[idx_vmem.at[0]], add=True)

        pltpu.emit_pipeline(
            body,
            grid=(N // SBLK,),
            in_specs=[
                pl.BlockSpec((SBLK, EXT), index_map=lambda i: (i, 0)),
                pl.BlockSpec((1, SBLK), index_map=lambda i: (i, 0)),
            ],
            out_specs=[],
            core_axis_name=("c", "s"),
            dimension_semantics=(pltpu.PARALLEL,),
        )(g_hbm, idx_hbm)

        plsc.subcore_barrier()
        pltpu.sync_copy(u_acc.at[pl.ds(sid * 624, 624)],
                        out_hbm.at[cid, pl.ds(sid * 624, 624)])

        @pl.when(sid == 15)
        def _():
            pltpu.sync_copy(u_acc.at[pl.ds(9984, 16)],
                            out_hbm.at[cid, pl.ds(9984, 16)])

    return k(g, idx2d, zeros_init)


def _finalize_body(u_ref, w3_ref, b3_ref, th_ref, rd_ref):
    u = u_ref[0] + u_ref[1]
    d = u[:, H:H + 1]
    rd = jnp.where(d > 0.0, 1.0 / d, 0.0)
    uh = u[:, :H] * rd
    th = jax.lax.dot_general(uh, w3_ref[...], (((1,), (0,)), ((), ())),
                             preferred_element_type=jnp.float32,
                             precision=jax.lax.Precision.HIGHEST)
    th_ref[...] = jnp.maximum(th + b3_ref[...], 0.0)
    rd_ref[...] = rd


def _finalize(u2, w3, b3r):
    return pl.pallas_call(
        _finalize_body,
        grid=(T // TBLK,),
        in_specs=[
            pl.BlockSpec((2, TBLK, EXT), lambda i: (0, i, 0)),
            pl.BlockSpec((H, H), lambda i: (0, 0)),
            pl.BlockSpec((1, H), lambda i: (0, 0)),
        ],
        out_specs=[
            pl.BlockSpec((TBLK, H), lambda i: (i, 0)),
            pl.BlockSpec((TBLK, 1), lambda i: (i, 0)),
        ],
        out_shape=[
            jax.ShapeDtypeStruct((T, H), jnp.float32),
            jax.ShapeDtypeStruct((T, 1), jnp.float32),
        ],
    )(u2, w3, b3r)


def _attn_gather(g, idx2d, rdenom):
    mesh = plsc.VectorSubcoreMesh(core_axis_name="c", subcore_axis_name="s")

    @functools.partial(
        pl.kernel,
        out_type=jax.ShapeDtypeStruct((N // SBLK, SBLK), jnp.float32),
        mesh=mesh,
        scratch_types=[pltpu.VMEM((T,), jnp.float32)],
        compiler_params=pltpu.CompilerParams(use_tc_tiling_on_sc=False,
                                             needs_layout_passes=False),
    )
    def k(g_hbm, idx_hbm, rd_hbm, out_hbm, rd_vmem):
        pltpu.sync_copy(rd_hbm, rd_vmem)

        def body(ex_vmem, idx_vmem, attn_vmem):
            lanes = jnp.arange(16, dtype=jnp.int32)
            zeros = jnp.zeros((16,), jnp.int32)
            for j in range(SBLK // 16):
                idxv = idx_vmem[0, pl.ds(j * 16, 16)]
                rd = plsc.load_gather(rd_vmem, [idxv])
                exv = plsc.load_gather(ex_vmem, [j * 16 + lanes, zeros])
                attn_vmem[0, pl.ds(j * 16, 16)] = exv * rd

        pltpu.emit_pipeline(
            body,
            grid=(N // SBLK,),
            in_specs=[
                pl.BlockSpec((SBLK, EXT - H), index_map=lambda i: (i, H // (EXT - H))),
                pl.BlockSpec((1, SBLK), index_map=lambda i: (i, 0)),
            ],
            out_specs=[pl.BlockSpec((1, SBLK), index_map=lambda i: (i, 0))],
            core_axis_name=("c", "s"),
            dimension_semantics=(pltpu.PARALLEL,),
        )(g_hbm, idx_hbm, out_hbm)

    return k(g, idx2d, rdenom)


def kernel(agent_h, team_idx, n_teams, W1, b1, W2, b2, W3, b3):
    idx2d = team_idx.astype(jnp.int32).reshape(N // SBLK, SBLK)
    scale2d = (jnp.asarray(n_teams, jnp.float32) / 10000.0).reshape(1, 1)
    b1r = b1.reshape(1, H // 2)
    w2r = W2.reshape(1, H // 2)
    b2r = b2.reshape(1, 1)
    b3r = b3.reshape(1, H)

    g = _scores_prescale(agent_h, W1, b1r, w2r, b2r, scale2d)
    u2 = _segment_accumulate(g, idx2d, jnp.zeros((T, EXT), jnp.float32))
    team_h, rd2d = _finalize(u2, W3, b3r)
    attn2d = _attn_gather(g, idx2d, rd2d.reshape(T))
    return team_h, attn2d.reshape(N)


# speedup vs baseline: 4.6978x; 1.2177x over previous
import functools

import jax
import jax.numpy as jnp
from jax import lax
from jax.experimental import pallas as pl
from jax.experimental.pallas import tpu as pltpu
from jax.experimental.pallas import tpu_sc as plsc

N = 100000
H = 128
T = 10000
EXT = 144
BLK1 = 2000
SBLK = 80
TBLK = 400
TPW = T // 16


def _scores_prescale_body(h_ref, w1_ref, b1_ref, w2_ref, b2_ref, sc_ref,
                          g_ref):
    h = h_ref[...]
    t1 = jnp.tanh(
        jax.lax.dot_general(h.astype(jnp.bfloat16),
                            w1_ref[...].astype(jnp.bfloat16),
                            (((1,), (0,)), ((), ())),
                            preferred_element_type=jnp.float32)
        + b1_ref[...])
    w2 = w2_ref[...]
    scale = sc_ref[0, 0]
    s = (jnp.sum(t1 * w2, axis=1, keepdims=True) + b2_ref[...]) * scale
    m = (jnp.sum(jnp.abs(w2)) + jnp.abs(b2_ref[0, 0])) * scale
    ex = jnp.exp(s - m)
    g_ref[:, :H] = h * ex
    g_ref[:, H:] = jnp.broadcast_to(ex, (BLK1, EXT - H))


def _scores_prescale(agent_h, w1, b1r, w2r, b2r, scale2d):
    return pl.pallas_call(
        _scores_prescale_body,
        grid=(N // BLK1,),
        in_specs=[
            pl.BlockSpec((BLK1, H), lambda i: (i, 0)),
            pl.BlockSpec((H, H // 2), lambda i: (0, 0)),
            pl.BlockSpec((1, H // 2), lambda i: (0, 0)),
            pl.BlockSpec((1, H // 2), lambda i: (0, 0)),
            pl.BlockSpec((1, 1), lambda i: (0, 0)),
            pl.BlockSpec((1, 1), lambda i: (0, 0)),
        ],
        out_specs=pl.BlockSpec((BLK1, EXT), lambda i: (i, 0)),
        out_shape=jax.ShapeDtypeStruct((N, EXT), jnp.float32),
    )(agent_h, w1, b1r, w2r, b2r, scale2d)


def _segment_accumulate(g, idx2d, zeros_init):
    mesh = plsc.VectorSubcoreMesh(core_axis_name="c", subcore_axis_name="s")

    @functools.partial(
        pl.kernel,
        out_type=jax.ShapeDtypeStruct((2, T, EXT), jnp.float32),
        mesh=mesh,
        scratch_types=[pltpu.VMEM_SHARED((T, EXT), jnp.float32)],
        compiler_params=pltpu.CompilerParams(use_tc_tiling_on_sc=False),
    )
    def k(g_hbm, idx_hbm, z_hbm, out_hbm, u_acc):
        cid = lax.axis_index("c")
        sid = lax.axis_index("s")
        pltpu.sync_copy(z_hbm.at[pl.ds(sid * 624, 624)],
                        u_acc.at[pl.ds(sid * 624, 624)])

        @pl.when(sid == 15)
        def _():
            pltpu.sync_copy(z_hbm.at[pl.ds(9984, 16)],
                            u_acc.at[pl.ds(9984, 16)])

        plsc.subcore_barrier()

        def body(g_vmem, idx_vmem):
            pltpu.sync_copy(g_vmem, u_acc.at[idx_vmem.at[0]], add=True)

        pltpu.emit_pipeline(
            body,
            grid=(N // SBLK,),
            in_specs=[
                pl.BlockSpec((SBLK, EXT), index_map=lambda i: (i, 0)),
                pl.BlockSpec((1, SBLK), index_map=lambda i: (i, 0)),
            ],
            out_specs=[],
            core_axis_name=("c", "s"),
            dimension_semantics=(pltpu.PARALLEL,),
        )(g_hbm, idx_hbm)

        plsc.subcore_barrier()
        pltpu.sync_copy(u_acc.at[pl.ds(sid * 624, 624)],
                        out_hbm.at[cid, pl.ds(sid * 624, 624)])

        @pl.when(sid == 15)
        def _():
            pltpu.sync_copy(u_acc.at[pl.ds(9984, 16)],
                            out_hbm.at[cid, pl.ds(9984, 16)])

    return k(g, idx2d, zeros_init)


def _finalize_body(u_ref, w3_ref, b3_ref, th_ref, rd_ref):
    u = u_ref[0] + u_ref[1]
    d = u[:, H:H + 1]
    rd = jnp.where(d > 0.0, 1.0 / d, 0.0)
    uh = u[:, :H] * rd
    th = jax.lax.dot_general(uh, w3_ref[...], (((1,), (0,)), ((), ())),
                             preferred_element_type=jnp.float32,
                             precision=jax.lax.Precision.HIGHEST)
    th_ref[...] = jnp.maximum(th + b3_ref[...], 0.0)
    rd_ref[...] = rd


def _finalize(u2, w3, b3r):
    return pl.pallas_call(
        _finalize_body,
        grid=(T // TBLK,),
        in_specs=[
            pl.BlockSpec((2, TBLK, EXT), lambda i: (0, i, 0)),
            pl.BlockSpec((H, H), lambda i: (0, 0)),
            pl.BlockSpec((1, H), lambda i: (0, 0)),
        ],
        out_specs=[
            pl.BlockSpec((TBLK, H), lambda i: (i, 0)),
            pl.BlockSpec((TBLK, 1), lambda i: (i, 0)),
        ],
        out_shape=[
            jax.ShapeDtypeStruct((T, H), jnp.float32),
            jax.ShapeDtypeStruct((T, 1), jnp.float32),
        ],
    )(u2, w3, b3r)


def _attn_gather(g, idx2d, rdenom):
    mesh = plsc.VectorSubcoreMesh(core_axis_name="c", subcore_axis_name="s")

    @functools.partial(
        pl.kernel,
        out_type=jax.ShapeDtypeStruct((N // SBLK, SBLK), jnp.float32),
        mesh=mesh,
        scratch_types=[pltpu.VMEM((T,), jnp.float32)],
        compiler_params=pltpu.CompilerParams(use_tc_tiling_on_sc=False,
                                             needs_layout_passes=False),
    )
    def k(g_hbm, idx_hbm, rd_hbm, out_hbm, rd_vmem):
        pltpu.sync_copy(rd_hbm, rd_vmem)

        def body(ex_vmem, idx_vmem, attn_vmem):
            lanes = jnp.arange(16, dtype=jnp.int32)
            zeros = jnp.zeros((16,), jnp.int32)
            for j in range(SBLK // 16):
                idxv = idx_vmem[0, pl.ds(j * 16, 16)]
                rd = plsc.load_gather(rd_vmem, [idxv])
                exv = plsc.load_gather(ex_vmem, [j * 16 + lanes, zeros])
                attn_vmem[0, pl.ds(j * 16, 16)] = exv * rd

        pltpu.emit_pipeline(
            body,
            grid=(N // SBLK,),
            in_specs=[
                pl.BlockSpec((SBLK, EXT - H), index_map=lambda i: (i, H // (EXT - H))),
                pl.BlockSpec((1, SBLK), index_map=lambda i: (i, 0)),
            ],
            out_specs=[pl.BlockSpec((1, SBLK), index_map=lambda i: (i, 0))],
            core_axis_name=("c", "s"),
            dimension_semantics=(pltpu.PARALLEL,),
        )(g_hbm, idx_hbm, out_hbm)

    return k(g, idx2d, rdenom)


def kernel(agent_h, team_idx, n_teams, W1, b1, W2, b2, W3, b3):
    idx2d = team_idx.astype(jnp.int32).reshape(N // SBLK, SBLK)
    scale2d = (jnp.asarray(n_teams, jnp.float32) / 10000.0).reshape(1, 1)
    b1r = b1.reshape(1, H // 2)
    w2r = W2.reshape(1, H // 2)
    b2r = b2.reshape(1, 1)
    b3r = b3.reshape(1, H)

    g = _scores_prescale(agent_h, W1, b1r, w2r, b2r, scale2d)
    u2 = _segment_accumulate(g, idx2d, jnp.zeros((T, EXT), jnp.float32))
    team_h, rd2d = _finalize(u2, W3, b3r)
    attn2d = _attn_gather(g, idx2d, rd2d.reshape(T))
    return team_h, attn2d.reshape(N)


# speedup vs baseline: 6.1607x; 1.3114x over previous
import functools

import jax
import jax.numpy as jnp
from jax import lax
from jax.experimental import pallas as pl
from jax.experimental.pallas import tpu as pltpu
from jax.experimental.pallas import tpu_sc as plsc

N = 100000
H = 128
T = 10000
DW = 16
BLK1 = 2000
SBLK = 80
TBLK = 400


def _scores_prescale_body(h_ref, w1_ref, b1_ref, w2_ref, b2_ref, sc_ref,
                          g_ref, ex_ref):
    h = h_ref[...]
    t1 = jnp.tanh(
        jax.lax.dot_general(h.astype(jnp.bfloat16),
                            w1_ref[...].astype(jnp.bfloat16),
                            (((1,), (0,)), ((), ())),
                            preferred_element_type=jnp.float32)
        + b1_ref[...])
    w2 = w2_ref[...]
    scale = sc_ref[0, 0]
    m = (jnp.sum(jnp.abs(w2)) + jnp.abs(b2_ref[0, 0])) * scale
    s_col = (jnp.sum(t1 * w2, axis=1, keepdims=True) + b2_ref[...]) * scale
    ex_col = jnp.exp(s_col - m)
    s_row = (jax.lax.dot_general(w2, t1, (((1,), (1,)), ((), ())),
                                 preferred_element_type=jnp.float32)
             + b2_ref[...]) * scale
    ex_row = jnp.exp(s_row - m)
    g_ref[...] = h * ex_col
    ex_ref[0] = ex_row


def _scores_prescale(agent_h, w1, b1r, w2r, b2r, scale2d):
    return pl.pallas_call(
        _scores_prescale_body,
        grid=(N // BLK1,),
        in_specs=[
            pl.BlockSpec((BLK1, H), lambda i: (i, 0)),
            pl.BlockSpec((H, H // 2), lambda i: (0, 0)),
            pl.BlockSpec((1, H // 2), lambda i: (0, 0)),
            pl.BlockSpec((1, H // 2), lambda i: (0, 0)),
            pl.BlockSpec((1, 1), lambda i: (0, 0)),
            pl.BlockSpec((1, 1), lambda i: (0, 0)),
        ],
        out_specs=[
            pl.BlockSpec((BLK1, H), lambda i: (i, 0)),
            pl.BlockSpec((1, 1, BLK1), lambda i: (i, 0, 0)),
        ],
        out_shape=[
            jax.ShapeDtypeStruct((N, H), jnp.float32),
            jax.ShapeDtypeStruct((N // BLK1, 1, BLK1), jnp.float32),
        ],
    )(agent_h, w1, b1r, w2r, b2r, scale2d)


def _segment_accumulate(g, idx2d, ex2d, z1, z2):
    mesh = plsc.VectorSubcoreMesh(core_axis_name="c", subcore_axis_name="s")

    @functools.partial(
        pl.kernel,
        out_type=[
            jax.ShapeDtypeStruct((2, T, H), jnp.float32),
            jax.ShapeDtypeStruct((2, T, DW), jnp.float32),
        ],
        mesh=mesh,
        scratch_types=[
            pltpu.VMEM_SHARED((T, H), jnp.float32),
            pltpu.VMEM_SHARED((T, DW), jnp.float32),
            pltpu.VMEM((SBLK, DW), jnp.float32),
        ],
        compiler_params=pltpu.CompilerParams(use_tc_tiling_on_sc=False,
                                             needs_layout_passes=False),
    )
    def k(g_hbm, idx_hbm, ex_hbm, z1_hbm, z2_hbm, out1_hbm, out2_hbm,
          acc1, acc2, exrow):
        cid = lax.axis_index("c")
        sid = lax.axis_index("s")
        pltpu.sync_copy(z1_hbm.at[pl.ds(sid * 624, 624)],
                        acc1.at[pl.ds(sid * 624, 624)])
        pltpu.sync_copy(z2_hbm.at[pl.ds(sid * 624, 624)],
                        acc2.at[pl.ds(sid * 624, 624)])

        @pl.when(sid == 15)
        def _():
            pltpu.sync_copy(z1_hbm.at[pl.ds(9984, 16)],
                            acc1.at[pl.ds(9984, 16)])
            pltpu.sync_copy(z2_hbm.at[pl.ds(9984, 16)],
                            acc2.at[pl.ds(9984, 16)])

        zero16 = jnp.zeros((16,), jnp.float32)

        @pl.loop(0, SBLK)
        def _(i):
            exrow[i, :] = zero16

        plsc.subcore_barrier()

        lanes = jnp.arange(16, dtype=jnp.int32)
        zlanes = jnp.zeros((16,), jnp.int32)

        def body(g_vmem, idx_vmem, ex_vmem):
            for j in range(SBLK // 16):
                plsc.store_scatter(exrow, [j * 16 + lanes, zlanes],
                                   ex_vmem[0, pl.ds(j * 16, 16)])
            pltpu.sync_copy(g_vmem, acc1.at[idx_vmem.at[0]], add=True)
            pltpu.sync_copy(exrow, acc2.at[idx_vmem.at[0]], add=True)

        pltpu.emit_pipeline(
            body,
            grid=(N // SBLK,),
            in_specs=[
                pl.BlockSpec((SBLK, H), index_map=lambda i: (i, 0)),
                pl.BlockSpec((1, SBLK), index_map=lambda i: (i, 0)),
                pl.BlockSpec((1, SBLK), index_map=lambda i: (i, 0)),
            ],
            out_specs=[],
            core_axis_name=("c", "s"),
            dimension_semantics=(pltpu.PARALLEL,),
        )(g_hbm, idx_hbm, ex_hbm)

        plsc.subcore_barrier()
        pltpu.sync_copy(acc1.at[pl.ds(sid * 624, 624)],
                        out1_hbm.at[cid, pl.ds(sid * 624, 624)])
        pltpu.sync_copy(acc2.at[pl.ds(sid * 624, 624)],
                        out2_hbm.at[cid, pl.ds(sid * 624, 624)])

        @pl.when(sid == 15)
        def _():
            pltpu.sync_copy(acc1.at[pl.ds(9984, 16)],
                            out1_hbm.at[cid, pl.ds(9984, 16)])
            pltpu.sync_copy(acc2.at[pl.ds(9984, 16)],
                            out2_hbm.at[cid, pl.ds(9984, 16)])

    return k(g, idx2d, ex2d, z1, z2)


def _finalize_body(uh_ref, ud_ref, w3_ref, b3_ref, th_ref, rd_ref):
    d = ud_ref[0, :, 0:1] + ud_ref[1, :, 0:1]
    rd = jnp.where(d > 0.0, 1.0 / d, 0.0)
    uh = (uh_ref[0] + uh_ref[1]) * rd
    th = jax.lax.dot_general(uh, w3_ref[...], (((1,), (0,)), ((), ())),
                             preferred_element_type=jnp.float32,
                             precision=jax.lax.Precision.HIGHEST)
    th_ref[...] = jnp.maximum(th + b3_ref[...], 0.0)
    rd_ref[...] = rd


def _finalize(u2h, u2d, w3, b3r):
    return pl.pallas_call(
        _finalize_body,
        grid=(T // TBLK,),
        in_specs=[
            pl.BlockSpec((2, TBLK, H), lambda i: (0, i, 0)),
            pl.BlockSpec((2, TBLK, DW), lambda i: (0, i, 0)),
            pl.BlockSpec((H, H), lambda i: (0, 0)),
            pl.BlockSpec((1, H), lambda i: (0, 0)),
        ],
        out_specs=[
            pl.BlockSpec((TBLK, H), lambda i: (i, 0)),
            pl.BlockSpec((TBLK, 1), lambda i: (i, 0)),
        ],
        out_shape=[
            jax.ShapeDtypeStruct((T, H), jnp.float32),
            jax.ShapeDtypeStruct((T, 1), jnp.float32),
        ],
    )(u2h, u2d, w3, b3r)


def _attn_gather(ex2d, idx2d, rdenom):
    mesh = plsc.VectorSubcoreMesh(core_axis_name="c", subcore_axis_name="s")

    @functools.partial(
        pl.kernel,
        out_type=jax.ShapeDtypeStruct((N // SBLK, SBLK), jnp.float32),
        mesh=mesh,
        scratch_types=[pltpu.VMEM((T,), jnp.float32)],
        compiler_params=pltpu.CompilerParams(use_tc_tiling_on_sc=False,
                                             needs_layout_passes=False),
    )
    def k(ex_hbm, idx_hbm, rd_hbm, out_hbm, rd_vmem):
        pltpu.sync_copy(rd_hbm, rd_vmem)

        def body(ex_vmem, idx_vmem, attn_vmem):
            for j in range(SBLK // 16):
                idxv = idx_vmem[0, pl.ds(j * 16, 16)]
                rd = plsc.load_gather(rd_vmem, [idxv])
                attn_vmem[0, pl.ds(j * 16, 16)] = (
                    ex_vmem[0, pl.ds(j * 16, 16)] * rd)

        pltpu.emit_pipeline(
            body,
            grid=(N // SBLK,),
            in_specs=[
                pl.BlockSpec((1, SBLK), index_map=lambda i: (i, 0)),
                pl.BlockSpec((1, SBLK), index_map=lambda i: (i, 0)),
            ],
            out_specs=[pl.BlockSpec((1, SBLK), index_map=lambda i: (i, 0))],
            core_axis_name=("c", "s"),
            dimension_semantics=(pltpu.PARALLEL,),
        )(ex_hbm, idx_hbm, out_hbm)

    return k(ex2d, idx2d, rdenom)


def kernel(agent_h, team_idx, n_teams, W1, b1, W2, b2, W3, b3):
    idx2d = team_idx.astype(jnp.int32).reshape(N // SBLK, SBLK)
    scale2d = (jnp.asarray(n_teams, jnp.float32) / 10000.0).reshape(1, 1)
    b1r = b1.reshape(1, H // 2)
    w2r = W2.reshape(1, H // 2)
    b2r = b2.reshape(1, 1)
    b3r = b3.reshape(1, H)

    g, ex3d = _scores_prescale(agent_h, W1, b1r, w2r, b2r, scale2d)
    ex2d = ex3d.reshape(N // SBLK, SBLK)
    u2h, u2d = _segment_accumulate(g, idx2d, ex2d,
                                   jnp.zeros((T, H), jnp.float32),
                                   jnp.zeros((T, DW), jnp.float32))
    team_h, rd2d = _finalize(u2h, u2d, W3, b3r)
    attn2d = _attn_gather(ex2d, idx2d, rd2d.reshape(T))
    return team_h, attn2d.reshape(N)


# speedup vs baseline: 8.0499x; 1.3066x over previous
import functools

import jax
import jax.numpy as jnp
from jax import lax
from jax.experimental import pallas as pl
from jax.experimental.pallas import tpu as pltpu
from jax.experimental.pallas import tpu_sc as plsc

N = 100000
H = 128
T = 10000
DW = 16
BLK1 = 2000
SBLK = 80
TBLK = 400


def _scores_prescale_body(h_ref, w1_ref, b1_ref, w2_ref, w2c_ref, b2_ref,
                          sc_ref, g_ref, ex_ref):
    h = h_ref[...]
    t1 = jnp.tanh(
        jax.lax.dot_general(h.astype(jnp.bfloat16),
                            w1_ref[...].astype(jnp.bfloat16),
                            (((1,), (0,)), ((), ())),
                            preferred_element_type=jnp.float32)
        + b1_ref[...])
    w2 = w2_ref[...]
    scale = sc_ref[0, 0]
    m = (jnp.sum(jnp.abs(w2)) + jnp.abs(b2_ref[0, 0])) * scale
    t1b = t1.astype(jnp.bfloat16)
    w2b = w2.astype(jnp.bfloat16)
    s_col = (jax.lax.dot_general(t1b, w2c_ref[...].astype(jnp.bfloat16),
                                 (((1,), (0,)), ((), ())),
                                 preferred_element_type=jnp.float32)
             + b2_ref[...]) * scale
    ex_col = jnp.exp(s_col - m)
    s_row = (jax.lax.dot_general(w2b, t1b, (((1,), (1,)), ((), ())),
                                 preferred_element_type=jnp.float32)
             + b2_ref[...]) * scale
    ex_row = jnp.exp(s_row - m)
    g_ref[...] = h * ex_col
    ex_ref[0] = ex_row


def _scores_prescale(agent_h, w1, b1r, w2r, w2c, b2r, scale2d):
    return pl.pallas_call(
        _scores_prescale_body,
        grid=(N // BLK1,),
        in_specs=[
            pl.BlockSpec((BLK1, H), lambda i: (i, 0)),
            pl.BlockSpec((H, H // 2), lambda i: (0, 0)),
            pl.BlockSpec((1, H // 2), lambda i: (0, 0)),
            pl.BlockSpec((1, H // 2), lambda i: (0, 0)),
            pl.BlockSpec((H // 2, 1), lambda i: (0, 0)),
            pl.BlockSpec((1, 1), lambda i: (0, 0)),
            pl.BlockSpec((1, 1), lambda i: (0, 0)),
        ],
        out_specs=[
            pl.BlockSpec((BLK1, H), lambda i: (i, 0)),
            pl.BlockSpec((1, 1, BLK1), lambda i: (i, 0, 0)),
        ],
        out_shape=[
            jax.ShapeDtypeStruct((N, H), jnp.float32),
            jax.ShapeDtypeStruct((N // BLK1, 1, BLK1), jnp.float32),
        ],
    )(agent_h, w1, b1r, w2r, w2c, b2r, scale2d)


def _segment_accumulate(g, idx2d, ex2d, z1, z2):
    mesh = plsc.VectorSubcoreMesh(core_axis_name="c", subcore_axis_name="s")

    @functools.partial(
        pl.kernel,
        out_type=[
            jax.ShapeDtypeStruct((2, T, H), jnp.float32),
            jax.ShapeDtypeStruct((2, T, DW), jnp.float32),
        ],
        mesh=mesh,
        scratch_types=[
            pltpu.VMEM_SHARED((T, H), jnp.float32),
            pltpu.VMEM_SHARED((T, DW), jnp.float32),
            pltpu.VMEM((SBLK, DW), jnp.float32),
        ],
        compiler_params=pltpu.CompilerParams(use_tc_tiling_on_sc=False,
                                             needs_layout_passes=False),
    )
    def k(g_hbm, idx_hbm, ex_hbm, z1_hbm, z2_hbm, out1_hbm, out2_hbm,
          acc1, acc2, exrow):
        cid = lax.axis_index("c")
        sid = lax.axis_index("s")
        pltpu.sync_copy(z1_hbm.at[pl.ds(sid * 624, 624)],
                        acc1.at[pl.ds(sid * 624, 624)])
        pltpu.sync_copy(z2_hbm.at[pl.ds(sid * 624, 624)],
                        acc2.at[pl.ds(sid * 624, 624)])

        @pl.when(sid == 15)
        def _():
            pltpu.sync_copy(z1_hbm.at[pl.ds(9984, 16)],
                            acc1.at[pl.ds(9984, 16)])
            pltpu.sync_copy(z2_hbm.at[pl.ds(9984, 16)],
                            acc2.at[pl.ds(9984, 16)])

        zero16 = jnp.zeros((16,), jnp.float32)

        @pl.loop(0, SBLK)
        def _(i):
            exrow[i, :] = zero16

        plsc.subcore_barrier()

        lanes = jnp.arange(16, dtype=jnp.int32)
        zlanes = jnp.zeros((16,), jnp.int32)

        def body(g_vmem, idx_vmem, ex_vmem):
            for j in range(SBLK // 16):
                plsc.store_scatter(exrow, [j * 16 + lanes, zlanes],
                                   ex_vmem[0, pl.ds(j * 16, 16)])
            pltpu.sync_copy(g_vmem, acc1.at[idx_vmem.at[0]], add=True)
            pltpu.sync_copy(exrow, acc2.at[idx_vmem.at[0]], add=True)

        pltpu.emit_pipeline(
            body,
            grid=(N // SBLK,),
            in_specs=[
                pl.BlockSpec((SBLK, H), index_map=lambda i: (i, 0)),
                pl.BlockSpec((1, SBLK), index_map=lambda i: (i, 0)),
                pl.BlockSpec((1, SBLK), index_map=lambda i: (i, 0)),
            ],
            out_specs=[],
            core_axis_name=("c", "s"),
            dimension_semantics=(pltpu.PARALLEL,),
        )(g_hbm, idx_hbm, ex_hbm)

        plsc.subcore_barrier()
        pltpu.sync_copy(acc1.at[pl.ds(sid * 624, 624)],
                        out1_hbm.at[cid, pl.ds(sid * 624, 624)])
        pltpu.sync_copy(acc2.at[pl.ds(sid * 624, 624)],
                        out2_hbm.at[cid, pl.ds(sid * 624, 624)])

        @pl.when(sid == 15)
        def _():
            pltpu.sync_copy(acc1.at[pl.ds(9984, 16)],
                            out1_hbm.at[cid, pl.ds(9984, 16)])
            pltpu.sync_copy(acc2.at[pl.ds(9984, 16)],
                            out2_hbm.at[cid, pl.ds(9984, 16)])

    return k(g, idx2d, ex2d, z1, z2)


def _finalize_body(uh_ref, ud_ref, w3_ref, b3_ref, th_ref, rd_ref):
    d = ud_ref[0, :, 0:1] + ud_ref[1, :, 0:1]
    rd = jnp.where(d > 0.0, 1.0 / d, 0.0)
    uh = (uh_ref[0] + uh_ref[1]) * rd
    th = jax.lax.dot_general(uh, w3_ref[...], (((1,), (0,)), ((), ())),
                             preferred_element_type=jnp.float32,
                             precision=jax.lax.Precision.HIGHEST)
    th_ref[...] = jnp.maximum(th + b3_ref[...], 0.0)
    rd_ref[...] = rd


def _finalize(u2h, u2d, w3, b3r):
    return pl.pallas_call(
        _finalize_body,
        grid=(T // TBLK,),
        in_specs=[
            pl.BlockSpec((2, TBLK, H), lambda i: (0, i, 0)),
            pl.BlockSpec((2, TBLK, DW), lambda i: (0, i, 0)),
            pl.BlockSpec((H, H), lambda i: (0, 0)),
            pl.BlockSpec((1, H), lambda i: (0, 0)),
        ],
        out_specs=[
            pl.BlockSpec((TBLK, H), lambda i: (i, 0)),
            pl.BlockSpec((TBLK, 1), lambda i: (i, 0)),
        ],
        out_shape=[
            jax.ShapeDtypeStruct((T, H), jnp.float32),
            jax.ShapeDtypeStruct((T, 1), jnp.float32),
        ],
    )(u2h, u2d, w3, b3r)


def _attn_gather(ex2d, idx2d, rdenom):
    mesh = plsc.VectorSubcoreMesh(core_axis_name="c", subcore_axis_name="s")

    @functools.partial(
        pl.kernel,
        out_type=jax.ShapeDtypeStruct((N // SBLK, SBLK), jnp.float32),
        mesh=mesh,
        scratch_types=[pltpu.VMEM((T,), jnp.float32)],
        compiler_params=pltpu.CompilerParams(use_tc_tiling_on_sc=False,
                                             needs_layout_passes=False),
    )
    def k(ex_hbm, idx_hbm, rd_hbm, out_hbm, rd_vmem):
        pltpu.sync_copy(rd_hbm, rd_vmem)

        def body(ex_vmem, idx_vmem, attn_vmem):
            for j in range(SBLK // 16):
                idxv = idx_vmem[0, pl.ds(j * 16, 16)]
                rd = plsc.load_gather(rd_vmem, [idxv])
                attn_vmem[0, pl.ds(j * 16, 16)] = (
                    ex_vmem[0, pl.ds(j * 16, 16)] * rd)

        pltpu.emit_pipeline(
            body,
            grid=(N // SBLK,),
            in_specs=[
                pl.BlockSpec((1, SBLK), index_map=lambda i: (i, 0)),
                pl.BlockSpec((1, SBLK), index_map=lambda i: (i, 0)),
            ],
            out_specs=[pl.BlockSpec((1, SBLK), index_map=lambda i: (i, 0))],
            core_axis_name=("c", "s"),
            dimension_semantics=(pltpu.PARALLEL,),
        )(ex_hbm, idx_hbm, out_hbm)

    return k(ex2d, idx2d, rdenom)


def kernel(agent_h, team_idx, n_teams, W1, b1, W2, b2, W3, b3):
    idx2d = team_idx.astype(jnp.int32).reshape(N // SBLK, SBLK)
    scale2d = (jnp.asarray(n_teams, jnp.float32) / 10000.0).reshape(1, 1)
    b1r = b1.reshape(1, H // 2)
    w2r = W2.reshape(1, H // 2)
    b2r = b2.reshape(1, 1)
    b3r = b3.reshape(1, H)

    g, ex3d = _scores_prescale(agent_h, W1, b1r, w2r, W2, b2r, scale2d)
    ex2d = ex3d.reshape(N // SBLK, SBLK)
    u2h, u2d = _segment_accumulate(g, idx2d, ex2d,
                                   jnp.zeros((T, H), jnp.float32),
                                   jnp.zeros((T, DW), jnp.float32))
    team_h, rd2d = _finalize(u2h, u2d, W3, b3r)
    attn2d = _attn_gather(ex2d, idx2d, rd2d.reshape(T))
    return team_h, attn2d.reshape(N)


# speedup vs baseline: 8.5830x; 1.0662x over previous
import functools

import jax
import jax.numpy as jnp
from jax import lax
from jax.experimental import pallas as pl
from jax.experimental.pallas import tpu as pltpu
from jax.experimental.pallas import tpu_sc as plsc

N = 100000
H = 128
T = 10000
DW = 16
BLK1 = 2000
SBLK = 80
TBLK = 2000


def _scores_prescale_body(h_ref, w1_ref, b1_ref, w2_ref, w2c_ref, b2_ref,
                          sc_ref, g_ref, ex_ref):
    h = h_ref[...]
    t1 = jnp.tanh(
        jax.lax.dot_general(h.astype(jnp.bfloat16),
                            w1_ref[...].astype(jnp.bfloat16),
                            (((1,), (0,)), ((), ())),
                            preferred_element_type=jnp.float32)
        + b1_ref[...])
    w2 = w2_ref[...]
    scale = sc_ref[0, 0]
    m = (jnp.sum(jnp.abs(w2)) + jnp.abs(b2_ref[0, 0])) * scale
    t1b = t1.astype(jnp.bfloat16)
    w2b = w2.astype(jnp.bfloat16)
    s_col = (jax.lax.dot_general(t1b, w2c_ref[...].astype(jnp.bfloat16),
                                 (((1,), (0,)), ((), ())),
                                 preferred_element_type=jnp.float32)
             + b2_ref[...]) * scale
    ex_col = jnp.exp(s_col - m)
    s_row = (jax.lax.dot_general(w2b, t1b, (((1,), (1,)), ((), ())),
                                 preferred_element_type=jnp.float32)
             + b2_ref[...]) * scale
    ex_row = jnp.exp(s_row - m)
    g_ref[...] = h * ex_col
    ex_ref[0] = ex_row


def _scores_prescale(agent_h, w1, b1r, w2r, w2c, b2r, scale2d):
    return pl.pallas_call(
        _scores_prescale_body,
        grid=(N // BLK1,),
        in_specs=[
            pl.BlockSpec((BLK1, H), lambda i: (i, 0)),
            pl.BlockSpec((H, H // 2), lambda i: (0, 0)),
            pl.BlockSpec((1, H // 2), lambda i: (0, 0)),
            pl.BlockSpec((1, H // 2), lambda i: (0, 0)),
            pl.BlockSpec((H // 2, 1), lambda i: (0, 0)),
            pl.BlockSpec((1, 1), lambda i: (0, 0)),
            pl.BlockSpec((1, 1), lambda i: (0, 0)),
        ],
        out_specs=[
            pl.BlockSpec((BLK1, H), lambda i: (i, 0)),
            pl.BlockSpec((1, 1, BLK1), lambda i: (i, 0, 0)),
        ],
        out_shape=[
            jax.ShapeDtypeStruct((N, H), jnp.float32),
            jax.ShapeDtypeStruct((N // BLK1, 1, BLK1), jnp.float32),
        ],
    )(agent_h, w1, b1r, w2r, w2c, b2r, scale2d)


def _segment_accumulate(g, idx2d, ex2d, z1, z2):
    mesh = plsc.VectorSubcoreMesh(core_axis_name="c", subcore_axis_name="s")

    @functools.partial(
        pl.kernel,
        out_type=[
            jax.ShapeDtypeStruct((2, T, H), jnp.float32),
            jax.ShapeDtypeStruct((2, T, DW), jnp.float32),
        ],
        mesh=mesh,
        scratch_types=[
            pltpu.VMEM_SHARED((T, H), jnp.float32),
            pltpu.VMEM_SHARED((T, DW), jnp.float32),
            pltpu.VMEM((SBLK, DW), jnp.float32),
        ],
        compiler_params=pltpu.CompilerParams(use_tc_tiling_on_sc=False,
                                             needs_layout_passes=False),
    )
    def k(g_hbm, idx_hbm, ex_hbm, z1_hbm, z2_hbm, out1_hbm, out2_hbm,
          acc1, acc2, exrow):
        cid = lax.axis_index("c")
        sid = lax.axis_index("s")
        pltpu.sync_copy(z1_hbm.at[pl.ds(sid * 624, 624)],
                        acc1.at[pl.ds(sid * 624, 624)])
        pltpu.sync_copy(z2_hbm.at[pl.ds(sid * 624, 624)],
                        acc2.at[pl.ds(sid * 624, 624)])

        @pl.when(sid == 15)
        def _():
            pltpu.sync_copy(z1_hbm.at[pl.ds(9984, 16)],
                            acc1.at[pl.ds(9984, 16)])
            pltpu.sync_copy(z2_hbm.at[pl.ds(9984, 16)],
                            acc2.at[pl.ds(9984, 16)])

        zero16 = jnp.zeros((16,), jnp.float32)

        @pl.loop(0, SBLK)
        def _(i):
            exrow[i, :] = zero16

        plsc.subcore_barrier()

        lanes = jnp.arange(16, dtype=jnp.int32)
        zlanes = jnp.zeros((16,), jnp.int32)

        def body(g_vmem, idx_vmem, ex_vmem):
            for j in range(SBLK // 16):
                plsc.store_scatter(exrow, [j * 16 + lanes, zlanes],
                                   ex_vmem[0, pl.ds(j * 16, 16)])
            pltpu.sync_copy(g_vmem, acc1.at[idx_vmem.at[0]], add=True)
            pltpu.sync_copy(exrow, acc2.at[idx_vmem.at[0]], add=True)

        pltpu.emit_pipeline(
            body,
            grid=(N // SBLK,),
            in_specs=[
                pl.BlockSpec((SBLK, H), index_map=lambda i: (i, 0)),
                pl.BlockSpec((1, SBLK), index_map=lambda i: (i, 0)),
                pl.BlockSpec((1, SBLK), index_map=lambda i: (i, 0)),
            ],
            out_specs=[],
            core_axis_name=("c", "s"),
            dimension_semantics=(pltpu.PARALLEL,),
        )(g_hbm, idx_hbm, ex_hbm)

        plsc.subcore_barrier()
        pltpu.sync_copy(acc1.at[pl.ds(sid * 624, 624)],
                        out1_hbm.at[cid, pl.ds(sid * 624, 624)])
        pltpu.sync_copy(acc2.at[pl.ds(sid * 624, 624)],
                        out2_hbm.at[cid, pl.ds(sid * 624, 624)])

        @pl.when(sid == 15)
        def _():
            pltpu.sync_copy(acc1.at[pl.ds(9984, 16)],
                            out1_hbm.at[cid, pl.ds(9984, 16)])
            pltpu.sync_copy(acc2.at[pl.ds(9984, 16)],
                            out2_hbm.at[cid, pl.ds(9984, 16)])

    return k(g, idx2d, ex2d, z1, z2)


def _finalize_body(uh_ref, ud_ref, w3_ref, b3_ref, th_ref, rd_ref):
    d = ud_ref[0, :, 0:1] + ud_ref[1, :, 0:1]
    rd = jnp.where(d > 0.0, 1.0 / d, 0.0)
    uh = (uh_ref[0] + uh_ref[1]) * rd
    th = jax.lax.dot_general(uh.astype(jnp.bfloat16),
                             w3_ref[...].astype(jnp.bfloat16),
                             (((1,), (0,)), ((), ())),
                             preferred_element_type=jnp.float32)
    th_ref[...] = jnp.maximum(th + b3_ref[...], 0.0)
    rd_ref[...] = rd


def _finalize(u2h, u2d, w3, b3r):
    return pl.pallas_call(
        _finalize_body,
        grid=(T // TBLK,),
        in_specs=[
            pl.BlockSpec((2, TBLK, H), lambda i: (0, i, 0)),
            pl.BlockSpec((2, TBLK, DW), lambda i: (0, i, 0)),
            pl.BlockSpec((H, H), lambda i: (0, 0)),
            pl.BlockSpec((1, H), lambda i: (0, 0)),
        ],
        out_specs=[
            pl.BlockSpec((TBLK, H), lambda i: (i, 0)),
            pl.BlockSpec((TBLK, 1), lambda i: (i, 0)),
        ],
        out_shape=[
            jax.ShapeDtypeStruct((T, H), jnp.float32),
            jax.ShapeDtypeStruct((T, 1), jnp.float32),
        ],
    )(u2h, u2d, w3, b3r)


def _attn_gather(ex2d, idx2d, rdenom):
    mesh = plsc.VectorSubcoreMesh(core_axis_name="c", subcore_axis_name="s")

    @functools.partial(
        pl.kernel,
        out_type=jax.ShapeDtypeStruct((N // SBLK, SBLK), jnp.float32),
        mesh=mesh,
        scratch_types=[pltpu.VMEM((T,), jnp.float32)],
        compiler_params=pltpu.CompilerParams(use_tc_tiling_on_sc=False,
                                             needs_layout_passes=False),
    )
    def k(ex_hbm, idx_hbm, rd_hbm, out_hbm, rd_vmem):
        pltpu.sync_copy(rd_hbm, rd_vmem)

        def body(ex_vmem, idx_vmem, attn_vmem):
            for j in range(SBLK // 16):
                idxv = idx_vmem[0, pl.ds(j * 16, 16)]
                rd = plsc.load_gather(rd_vmem, [idxv])
                attn_vmem[0, pl.ds(j * 16, 16)] = (
                    ex_vmem[0, pl.ds(j * 16, 16)] * rd)

        pltpu.emit_pipeline(
            body,
            grid=(N // SBLK,),
            in_specs=[
                pl.BlockSpec((1, SBLK), index_map=lambda i: (i, 0)),
                pl.BlockSpec((1, SBLK), index_map=lambda i: (i, 0)),
            ],
            out_specs=[pl.BlockSpec((1, SBLK), index_map=lambda i: (i, 0))],
            core_axis_name=("c", "s"),
            dimension_semantics=(pltpu.PARALLEL,),
        )(ex_hbm, idx_hbm, out_hbm)

    return k(ex2d, idx2d, rdenom)


def kernel(agent_h, team_idx, n_teams, W1, b1, W2, b2, W3, b3):
    idx2d = team_idx.astype(jnp.int32).reshape(N // SBLK, SBLK)
    scale2d = (jnp.asarray(n_teams, jnp.float32) / 10000.0).reshape(1, 1)
    b1r = b1.reshape(1, H // 2)
    w2r = W2.reshape(1, H // 2)
    b2r = b2.reshape(1, 1)
    b3r = b3.reshape(1, H)

    g, ex3d = _scores_prescale(agent_h, W1, b1r, w2r, W2, b2r, scale2d)
    ex2d = ex3d.reshape(N // SBLK, SBLK)
    u2h, u2d = _segment_accumulate(g, idx2d, ex2d,
                                   jnp.zeros((T, H), jnp.float32),
                                   jnp.zeros((T, DW), jnp.float32))
    team_h, rd2d = _finalize(u2h, u2d, W3, b3r)
    attn2d = _attn_gather(ex2d, idx2d, rd2d.reshape(T))
    return team_h, attn2d.reshape(N)


# speedup vs baseline: 10.0680x; 1.1730x over previous
import functools

import jax
import jax.numpy as jnp
from jax import lax
from jax.experimental import pallas as pl
from jax.experimental.pallas import tpu as pltpu
from jax.experimental.pallas import tpu_sc as plsc

N = 100000
H = 128
T = 10000
DW = 16
BLK1 = 4000
SBLK = 80
TBLK = 2000


def _scores_prescale_body(h_ref, w1_ref, b1_ref, w2_ref, w2c_ref, b2_ref,
                          sc_ref, g_ref, ex_ref):
    h = h_ref[...]
    t1 = jnp.tanh(
        jax.lax.dot_general(h.astype(jnp.bfloat16),
                            w1_ref[...].astype(jnp.bfloat16),
                            (((1,), (0,)), ((), ())),
                            preferred_element_type=jnp.float32)
        + b1_ref[...])
    w2 = w2_ref[...]
    scale = sc_ref[0, 0]
    m = (jnp.sum(jnp.abs(w2)) + jnp.abs(b2_ref[0, 0])) * scale
    t1b = t1.astype(jnp.bfloat16)
    w2b = w2.astype(jnp.bfloat16)
    s_col = (jax.lax.dot_general(t1b, w2c_ref[...].astype(jnp.bfloat16),
                                 (((1,), (0,)), ((), ())),
                                 preferred_element_type=jnp.float32)
             + b2_ref[...]) * scale
    ex_col = jnp.exp(s_col - m)
    s_row = (jax.lax.dot_general(w2b, t1b, (((1,), (1,)), ((), ())),
                                 preferred_element_type=jnp.float32)
             + b2_ref[...]) * scale
    ex_row = jnp.exp(s_row - m)
    g_ref[...] = h * ex_col
    ex_ref[0] = ex_row


def _scores_prescale(agent_h, w1, b1r, w2r, w2c, b2r, scale2d):
    return pl.pallas_call(
        _scores_prescale_body,
        grid=(N // BLK1,),
        in_specs=[
            pl.BlockSpec((BLK1, H), lambda i: (i, 0)),
            pl.BlockSpec((H, H // 2), lambda i: (0, 0)),
            pl.BlockSpec((1, H // 2), lambda i: (0, 0)),
            pl.BlockSpec((1, H // 2), lambda i: (0, 0)),
            pl.BlockSpec((H // 2, 1), lambda i: (0, 0)),
            pl.BlockSpec((1, 1), lambda i: (0, 0)),
            pl.BlockSpec((1, 1), lambda i: (0, 0)),
        ],
        out_specs=[
            pl.BlockSpec((BLK1, H), lambda i: (i, 0)),
            pl.BlockSpec((1, 1, BLK1), lambda i: (i, 0, 0)),
        ],
        out_shape=[
            jax.ShapeDtypeStruct((N, H), jnp.float32),
            jax.ShapeDtypeStruct((N // BLK1, 1, BLK1), jnp.float32),
        ],
    )(agent_h, w1, b1r, w2r, w2c, b2r, scale2d)


def _segment_accumulate(g, idx2d, ex2d):
    mesh = plsc.VectorSubcoreMesh(core_axis_name="c", subcore_axis_name="s")

    @functools.partial(
        pl.kernel,
        out_type=[
            jax.ShapeDtypeStruct((2, T, H), jnp.float32),
            jax.ShapeDtypeStruct((2, T, DW), jnp.float32),
        ],
        mesh=mesh,
        scratch_types=[
            pltpu.VMEM_SHARED((T, H), jnp.float32),
            pltpu.VMEM_SHARED((T, DW), jnp.float32),
            pltpu.VMEM((SBLK, DW), jnp.float32),
            pltpu.VMEM((78, H), jnp.float32),
            pltpu.VMEM((78, DW), jnp.float32),
        ],
        compiler_params=pltpu.CompilerParams(use_tc_tiling_on_sc=False,
                                             needs_layout_passes=False),
    )
    def k(g_hbm, idx_hbm, ex_hbm, out1_hbm, out2_hbm,
          acc1, acc2, exrow, zbuf, zbuf2):
        cid = lax.axis_index("c")
        sid = lax.axis_index("s")
        zero16 = jnp.zeros((16,), jnp.float32)

        @pl.loop(0, 78)
        def _(i):
            for c in range(H // 16):
                zbuf[i, pl.ds(c * 16, 16)] = zero16
            zbuf2[i, :] = zero16

        @pl.loop(0, 8)
        def _(kk):
            pltpu.sync_copy(zbuf, acc1.at[pl.ds(sid * 624 + kk * 78, 78)])
            pltpu.sync_copy(zbuf2, acc2.at[pl.ds(sid * 624 + kk * 78, 78)])

        @pl.when(sid == 15)
        def _():
            pltpu.sync_copy(zbuf.at[pl.ds(0, 16)], acc1.at[pl.ds(9984, 16)])
            pltpu.sync_copy(zbuf2.at[pl.ds(0, 16)], acc2.at[pl.ds(9984, 16)])

        @pl.loop(0, SBLK)
        def _(i):
            exrow[i, :] = zero16

        plsc.subcore_barrier()

        lanes = jnp.arange(16, dtype=jnp.int32)
        zlanes = jnp.zeros((16,), jnp.int32)

        def body(g_vmem, idx_vmem, ex_vmem):
            for j in range(SBLK // 16):
                plsc.store_scatter(exrow, [j * 16 + lanes, zlanes],
                                   ex_vmem[0, pl.ds(j * 16, 16)])
            pltpu.sync_copy(g_vmem, acc1.at[idx_vmem.at[0]], add=True)
            pltpu.sync_copy(exrow, acc2.at[idx_vmem.at[0]], add=True)

        pltpu.emit_pipeline(
            body,
            grid=(N // SBLK,),
            in_specs=[
                pl.BlockSpec((SBLK, H), index_map=lambda i: (i, 0)),
                pl.BlockSpec((1, SBLK), index_map=lambda i: (i, 0)),
                pl.BlockSpec((1, SBLK), index_map=lambda i: (i, 0)),
            ],
            out_specs=[],
            core_axis_name=("c", "s"),
            dimension_semantics=(pltpu.PARALLEL,),
        )(g_hbm, idx_hbm, ex_hbm)

        plsc.subcore_barrier()
        pltpu.sync_copy(acc1.at[pl.ds(sid * 624, 624)],
                        out1_hbm.at[cid, pl.ds(sid * 624, 624)])
        pltpu.sync_copy(acc2.at[pl.ds(sid * 624, 624)],
                        out2_hbm.at[cid, pl.ds(sid * 624, 624)])

        @pl.when(sid == 15)
        def _():
            pltpu.sync_copy(acc1.at[pl.ds(9984, 16)],
                            out1_hbm.at[cid, pl.ds(9984, 16)])
            pltpu.sync_copy(acc2.at[pl.ds(9984, 16)],
                            out2_hbm.at[cid, pl.ds(9984, 16)])

    return k(g, idx2d, ex2d)


def _finalize_body(uh_ref, ud_ref, w3_ref, b3_ref, th_ref, rd_ref):
    d = ud_ref[0, :, 0:1] + ud_ref[1, :, 0:1]
    rd = jnp.where(d > 0.0, 1.0 / d, 0.0)
    uh = (uh_ref[0] + uh_ref[1]) * rd
    th = jax.lax.dot_general(uh.astype(jnp.bfloat16),
                             w3_ref[...].astype(jnp.bfloat16),
                             (((1,), (0,)), ((), ())),
                             preferred_element_type=jnp.float32)
    th_ref[...] = jnp.maximum(th + b3_ref[...], 0.0)
    rd_ref[...] = rd


def _finalize(u2h, u2d, w3, b3r):
    return pl.pallas_call(
        _finalize_body,
        grid=(T // TBLK,),
        in_specs=[
            pl.BlockSpec((2, TBLK, H), lambda i: (0, i, 0)),
            pl.BlockSpec((2, TBLK, DW), lambda i: (0, i, 0)),
            pl.BlockSpec((H, H), lambda i: (0, 0)),
            pl.BlockSpec((1, H), lambda i: (0, 0)),
        ],
        out_specs=[
            pl.BlockSpec((TBLK, H), lambda i: (i, 0)),
            pl.BlockSpec((TBLK, 1), lambda i: (i, 0)),
        ],
        out_shape=[
            jax.ShapeDtypeStruct((T, H), jnp.float32),
            jax.ShapeDtypeStruct((T, 1), jnp.float32),
        ],
    )(u2h, u2d, w3, b3r)


def _attn_gather(ex2d, idx2d, rdenom):
    mesh = plsc.VectorSubcoreMesh(core_axis_name="c", subcore_axis_name="s")

    @functools.partial(
        pl.kernel,
        out_type=jax.ShapeDtypeStruct((N // SBLK, SBLK), jnp.float32),
        mesh=mesh,
        scratch_types=[pltpu.VMEM((T,), jnp.float32)],
        compiler_params=pltpu.CompilerParams(use_tc_tiling_on_sc=False,
                                             needs_layout_passes=False),
    )
    def k(ex_hbm, idx_hbm, rd_hbm, out_hbm, rd_vmem):
        pltpu.sync_copy(rd_hbm, rd_vmem)

        def body(ex_vmem, idx_vmem, attn_vmem):
            for j in range(SBLK // 16):
                idxv = idx_vmem[0, pl.ds(j * 16, 16)]
                rd = plsc.load_gather(rd_vmem, [idxv])
                attn_vmem[0, pl.ds(j * 16, 16)] = (
                    ex_vmem[0, pl.ds(j * 16, 16)] * rd)

        pltpu.emit_pipeline(
            body,
            grid=(N // SBLK,),
            in_specs=[
                pl.BlockSpec((1, SBLK), index_map=lambda i: (i, 0)),
                pl.BlockSpec((1, SBLK), index_map=lambda i: (i, 0)),
            ],
            out_specs=[pl.BlockSpec((1, SBLK), index_map=lambda i: (i, 0))],
            core_axis_name=("c", "s"),
            dimension_semantics=(pltpu.PARALLEL,),
        )(ex_hbm, idx_hbm, out_hbm)

    return k(ex2d, idx2d, rdenom)


def kernel(agent_h, team_idx, n_teams, W1, b1, W2, b2, W3, b3):
    idx2d = team_idx.astype(jnp.int32).reshape(N // SBLK, SBLK)
    scale2d = (jnp.asarray(n_teams, jnp.float32) / 10000.0).reshape(1, 1)
    b1r = b1.reshape(1, H // 2)
    w2r = W2.reshape(1, H // 2)
    b2r = b2.reshape(1, 1)
    b3r = b3.reshape(1, H)

    g, ex3d = _scores_prescale(agent_h, W1, b1r, w2r, W2, b2r, scale2d)
    ex2d = ex3d.reshape(N // SBLK, SBLK)
    u2h, u2d = _segment_accumulate(g, idx2d, ex2d)
    team_h, rd2d = _finalize(u2h, u2d, W3, b3r)
    attn2d = _attn_gather(ex2d, idx2d, rd2d.reshape(T))
    return team_h, attn2d.reshape(N)


# speedup vs baseline: 10.1826x; 1.0114x over previous
import functools

import jax
import jax.numpy as jnp
from jax import lax
from jax.experimental import pallas as pl
from jax.experimental.pallas import tpu as pltpu
from jax.experimental.pallas import tpu_sc as plsc

N = 100000
H = 128
T = 10000
DW = 16
BLK1 = 4000
SBLK = 80
TBLK = 2000


def _scores_prescale_body(h_ref, w1_ref, b1_ref, w2_ref, w2c_ref, b2_ref,
                          sc_ref, g_ref, ex_ref):
    h = h_ref[...]
    t1 = jnp.tanh(
        jax.lax.dot_general(h.astype(jnp.bfloat16),
                            w1_ref[...].astype(jnp.bfloat16),
                            (((1,), (0,)), ((), ())),
                            preferred_element_type=jnp.float32)
        + b1_ref[...])
    w2 = w2_ref[...]
    scale = sc_ref[0, 0]
    m = (jnp.sum(jnp.abs(w2)) + jnp.abs(b2_ref[0, 0])) * scale
    t1b = t1.astype(jnp.bfloat16)
    w2b = w2.astype(jnp.bfloat16)
    s_col = (jax.lax.dot_general(t1b, w2c_ref[...].astype(jnp.bfloat16),
                                 (((1,), (0,)), ((), ())),
                                 preferred_element_type=jnp.float32)
             + b2_ref[...]) * scale
    ex_col = jnp.exp(s_col - m)
    s_row = (jax.lax.dot_general(w2b, t1b, (((1,), (1,)), ((), ())),
                                 preferred_element_type=jnp.float32)
             + b2_ref[...]) * scale
    ex_row = jnp.exp(s_row - m)
    g_ref[...] = h * ex_col
    ex_ref[0] = ex_row


def _scores_prescale(agent_h, w1, b1r, w2r, w2c, b2r, scale2d):
    return pl.pallas_call(
        _scores_prescale_body,
        grid=(N // BLK1,),
        in_specs=[
            pl.BlockSpec((BLK1, H), lambda i: (i, 0)),
            pl.BlockSpec((H, H // 2), lambda i: (0, 0)),
            pl.BlockSpec((1, H // 2), lambda i: (0, 0)),
            pl.BlockSpec((1, H // 2), lambda i: (0, 0)),
            pl.BlockSpec((H // 2, 1), lambda i: (0, 0)),
            pl.BlockSpec((1, 1), lambda i: (0, 0)),
            pl.BlockSpec((1, 1), lambda i: (0, 0)),
        ],
        out_specs=[
            pl.BlockSpec((BLK1, H), lambda i: (i, 0)),
            pl.BlockSpec((1, 1, BLK1), lambda i: (i, 0, 0)),
        ],
        out_shape=[
            jax.ShapeDtypeStruct((N, H), jnp.float32),
            jax.ShapeDtypeStruct((N // BLK1, 1, BLK1), jnp.float32),
        ],
    )(agent_h, w1, b1r, w2r, w2c, b2r, scale2d)


def _segment_accumulate(g, idx2d, ex2d):
    mesh = plsc.VectorSubcoreMesh(core_axis_name="c", subcore_axis_name="s")

    @functools.partial(
        pl.kernel,
        out_type=[
            jax.ShapeDtypeStruct((2, T, H), jnp.float32),
            jax.ShapeDtypeStruct((2, T, DW), jnp.float32),
        ],
        mesh=mesh,
        scratch_types=[
            pltpu.VMEM_SHARED((T, H), jnp.float32),
            pltpu.VMEM_SHARED((T, DW), jnp.float32),
            pltpu.VMEM((SBLK, DW), jnp.float32),
            pltpu.VMEM((78, H), jnp.float32),
            pltpu.VMEM((78, DW), jnp.float32),
        ],
        compiler_params=pltpu.CompilerParams(use_tc_tiling_on_sc=False,
                                             needs_layout_passes=False),
    )
    def k(g_hbm, idx_hbm, ex_hbm, out1_hbm, out2_hbm,
          acc1, acc2, exrow, zbuf, zbuf2):
        cid = lax.axis_index("c")
        sid = lax.axis_index("s")
        zero16 = jnp.zeros((16,), jnp.float32)

        @pl.loop(0, 78)
        def _(i):
            for c in range(H // 16):
                zbuf[i, pl.ds(c * 16, 16)] = zero16
            zbuf2[i, :] = zero16

        @pl.loop(0, 8)
        def _(kk):
            pltpu.sync_copy(zbuf, acc1.at[pl.ds(sid * 624 + kk * 78, 78)])
            pltpu.sync_copy(zbuf2, acc2.at[pl.ds(sid * 624 + kk * 78, 78)])

        @pl.when(sid == 15)
        def _():
            pltpu.sync_copy(zbuf.at[pl.ds(0, 16)], acc1.at[pl.ds(9984, 16)])
            pltpu.sync_copy(zbuf2.at[pl.ds(0, 16)], acc2.at[pl.ds(9984, 16)])

        @pl.loop(0, SBLK)
        def _(i):
            exrow[i, :] = zero16

        plsc.subcore_barrier()

        lanes = jnp.arange(16, dtype=jnp.int32)
        zlanes = jnp.zeros((16,), jnp.int32)

        def body(g_vmem, idx_vmem, ex_vmem):
            for j in range(SBLK // 16):
                plsc.store_scatter(exrow, [j * 16 + lanes, zlanes],
                                   ex_vmem[0, pl.ds(j * 16, 16)])
            pltpu.sync_copy(g_vmem, acc1.at[idx_vmem.at[0]], add=True)
            pltpu.sync_copy(exrow, acc2.at[idx_vmem.at[0]], add=True)

        pltpu.emit_pipeline(
            body,
            grid=(N // SBLK,),
            in_specs=[
                pl.BlockSpec((SBLK, H), index_map=lambda i: (i, 0)),
                pl.BlockSpec((1, SBLK), index_map=lambda i: (i, 0)),
                pl.BlockSpec((1, SBLK), index_map=lambda i: (i, 0)),
            ],
            out_specs=[],
            core_axis_name=("c", "s"),
            dimension_semantics=(pltpu.PARALLEL,),
        )(g_hbm, idx_hbm, ex_hbm)

        plsc.subcore_barrier()
        pltpu.sync_copy(acc1.at[pl.ds(sid * 624, 624)],
                        out1_hbm.at[cid, pl.ds(sid * 624, 624)])
        pltpu.sync_copy(acc2.at[pl.ds(sid * 624, 624)],
                        out2_hbm.at[cid, pl.ds(sid * 624, 624)])

        @pl.when(sid == 15)
        def _():
            pltpu.sync_copy(acc1.at[pl.ds(9984, 16)],
                            out1_hbm.at[cid, pl.ds(9984, 16)])
            pltpu.sync_copy(acc2.at[pl.ds(9984, 16)],
                            out2_hbm.at[cid, pl.ds(9984, 16)])

    return k(g, idx2d, ex2d)


def _rdenom_body(ud_ref, rd_ref):
    d = ud_ref[0, :, 0:1] + ud_ref[1, :, 0:1]
    rd_ref[...] = jnp.where(d > 0.0, 1.0 / d, 0.0)


def _rdenom(u2d):
    return pl.pallas_call(
        _rdenom_body,
        grid=(1,),
        in_specs=[pl.BlockSpec((2, T, DW), lambda i: (0, 0, 0))],
        out_specs=pl.BlockSpec((T, 1), lambda i: (0, 0)),
        out_shape=jax.ShapeDtypeStruct((T, 1), jnp.float32),
    )(u2d)


def _project_body(uh_ref, rd_ref, w3_ref, b3_ref, th_ref):
    uh = (uh_ref[0] + uh_ref[1]) * rd_ref[...]
    th = jax.lax.dot_general(uh.astype(jnp.bfloat16),
                             w3_ref[...].astype(jnp.bfloat16),
                             (((1,), (0,)), ((), ())),
                             preferred_element_type=jnp.float32)
    th_ref[...] = jnp.maximum(th + b3_ref[...], 0.0)


def _project(u2h, rd2d, w3, b3r):
    return pl.pallas_call(
        _project_body,
        grid=(T // TBLK,),
        in_specs=[
            pl.BlockSpec((2, TBLK, H), lambda i: (0, i, 0)),
            pl.BlockSpec((TBLK, 1), lambda i: (i, 0)),
            pl.BlockSpec((H, H), lambda i: (0, 0)),
            pl.BlockSpec((1, H), lambda i: (0, 0)),
        ],
        out_specs=pl.BlockSpec((TBLK, H), lambda i: (i, 0)),
        out_shape=jax.ShapeDtypeStruct((T, H), jnp.float32),
    )(u2h, rd2d, w3, b3r)


def _attn_gather(ex2d, idx2d, rdenom):
    mesh = plsc.VectorSubcoreMesh(core_axis_name="c", subcore_axis_name="s")

    @functools.partial(
        pl.kernel,
        out_type=jax.ShapeDtypeStruct((N // SBLK, SBLK), jnp.float32),
        mesh=mesh,
        scratch_types=[pltpu.VMEM((T,), jnp.float32)],
        compiler_params=pltpu.CompilerParams(use_tc_tiling_on_sc=False,
                                             needs_layout_passes=False),
    )
    def k(ex_hbm, idx_hbm, rd_hbm, out_hbm, rd_vmem):
        pltpu.sync_copy(rd_hbm, rd_vmem)

        def body(ex_vmem, idx_vmem, attn_vmem):
            for j in range(SBLK // 16):
                idxv = idx_vmem[0, pl.ds(j * 16, 16)]
                rd = plsc.load_gather(rd_vmem, [idxv])
                attn_vmem[0, pl.ds(j * 16, 16)] = (
                    ex_vmem[0, pl.ds(j * 16, 16)] * rd)

        pltpu.emit_pipeline(
            body,
            grid=(N // SBLK,),
            in_specs=[
                pl.BlockSpec((1, SBLK), index_map=lambda i: (i, 0)),
                pl.BlockSpec((1, SBLK), index_map=lambda i: (i, 0)),
            ],
            out_specs=[pl.BlockSpec((1, SBLK), index_map=lambda i: (i, 0))],
            core_axis_name=("c", "s"),
            dimension_semantics=(pltpu.PARALLEL,),
        )(ex_hbm, idx_hbm, out_hbm)

    return k(ex2d, idx2d, rdenom)


def kernel(agent_h, team_idx, n_teams, W1, b1, W2, b2, W3, b3):
    idx2d = team_idx.astype(jnp.int32).reshape(N // SBLK, SBLK)
    scale2d = (jnp.asarray(n_teams, jnp.float32) / 10000.0).reshape(1, 1)
    b1r = b1.reshape(1, H // 2)
    w2r = W2.reshape(1, H // 2)
    b2r = b2.reshape(1, 1)
    b3r = b3.reshape(1, H)

    g, ex3d = _scores_prescale(agent_h, W1, b1r, w2r, W2, b2r, scale2d)
    ex2d = ex3d.reshape(N // SBLK, SBLK)
    u2h, u2d = _segment_accumulate(g, idx2d, ex2d)
    rd2d = _rdenom(u2d)
    team_h = _project(u2h, rd2d, W3, b3r)
    attn2d = _attn_gather(ex2d, idx2d, rd2d.reshape(T))
    return team_h, attn2d.reshape(N)


# speedup vs baseline: 10.5612x; 1.0372x over previous
import functools

import jax
import jax.numpy as jnp
from jax import lax
from jax.experimental import pallas as pl
from jax.experimental.pallas import tpu as pltpu
from jax.experimental.pallas import tpu_sc as plsc

N = 100000
H = 128
T = 10000
DW = 16
BLK1 = 10000
SBLK = 80
TBLK = 2000


def _scores_prescale_body(h_ref, w1_ref, b1_ref, w2_ref, w2c_ref, b2_ref,
                          sc_ref, g_ref, ex_ref):
    h = h_ref[...]
    t1 = jnp.tanh(
        jax.lax.dot_general(h.astype(jnp.bfloat16),
                            w1_ref[...].astype(jnp.bfloat16),
                            (((1,), (0,)), ((), ())),
                            preferred_element_type=jnp.float32)
        + b1_ref[...])
    w2 = w2_ref[...]
    scale = sc_ref[0, 0]
    m = (jnp.sum(jnp.abs(w2)) + jnp.abs(b2_ref[0, 0])) * scale
    t1b = t1.astype(jnp.bfloat16)
    w2b = w2.astype(jnp.bfloat16)
    s_col = (jax.lax.dot_general(t1b, w2c_ref[...].astype(jnp.bfloat16),
                                 (((1,), (0,)), ((), ())),
                                 preferred_element_type=jnp.float32)
             + b2_ref[...]) * scale
    ex_col = jnp.exp(s_col - m)
    s_row = (jax.lax.dot_general(w2b, t1b, (((1,), (1,)), ((), ())),
                                 preferred_element_type=jnp.float32)
             + b2_ref[...]) * scale
    ex_row = jnp.exp(s_row - m)
    g_ref[...] = h * ex_col
    ex_ref[0] = ex_row


def _scores_prescale(agent_h, w1, b1r, w2r, w2c, b2r, scale2d):
    return pl.pallas_call(
        _scores_prescale_body,
        grid=(N // BLK1,),
        in_specs=[
            pl.BlockSpec((BLK1, H), lambda i: (i, 0)),
            pl.BlockSpec((H, H // 2), lambda i: (0, 0)),
            pl.BlockSpec((1, H // 2), lambda i: (0, 0)),
            pl.BlockSpec((1, H // 2), lambda i: (0, 0)),
            pl.BlockSpec((H // 2, 1), lambda i: (0, 0)),
            pl.BlockSpec((1, 1), lambda i: (0, 0)),
            pl.BlockSpec((1, 1), lambda i: (0, 0)),
        ],
        out_specs=[
            pl.BlockSpec((BLK1, H), lambda i: (i, 0)),
            pl.BlockSpec((1, 1, BLK1), lambda i: (i, 0, 0)),
        ],
        out_shape=[
            jax.ShapeDtypeStruct((N, H), jnp.float32),
            jax.ShapeDtypeStruct((N // BLK1, 1, BLK1), jnp.float32),
        ],
    )(agent_h, w1, b1r, w2r, w2c, b2r, scale2d)


def _segment_accumulate(g, idx2d, ex2d):
    mesh = plsc.VectorSubcoreMesh(core_axis_name="c", subcore_axis_name="s")

    @functools.partial(
        pl.kernel,
        out_type=[
            jax.ShapeDtypeStruct((2, T, H), jnp.float32),
            jax.ShapeDtypeStruct((2, T, DW), jnp.float32),
        ],
        mesh=mesh,
        scratch_types=[
            pltpu.VMEM_SHARED((T, H), jnp.float32),
            pltpu.VMEM_SHARED((T, DW), jnp.float32),
            pltpu.VMEM((SBLK, DW), jnp.float32),
            pltpu.VMEM((78, H), jnp.float32),
            pltpu.VMEM((78, DW), jnp.float32),
        ],
        compiler_params=pltpu.CompilerParams(use_tc_tiling_on_sc=False,
                                             needs_layout_passes=False),
    )
    def k(g_hbm, idx_hbm, ex_hbm, out1_hbm, out2_hbm,
          acc1, acc2, exrow, zbuf, zbuf2):
        cid = lax.axis_index("c")
        sid = lax.axis_index("s")
        zero16 = jnp.zeros((16,), jnp.float32)

        @pl.loop(0, 78)
        def _(i):
            for c in range(H // 16):
                zbuf[i, pl.ds(c * 16, 16)] = zero16
            zbuf2[i, :] = zero16

        @pl.loop(0, 8)
        def _(kk):
            pltpu.sync_copy(zbuf, acc1.at[pl.ds(sid * 624 + kk * 78, 78)])
            pltpu.sync_copy(zbuf2, acc2.at[pl.ds(sid * 624 + kk * 78, 78)])

        @pl.when(sid == 15)
        def _():
            pltpu.sync_copy(zbuf.at[pl.ds(0, 16)], acc1.at[pl.ds(9984, 16)])
            pltpu.sync_copy(zbuf2.at[pl.ds(0, 16)], acc2.at[pl.ds(9984, 16)])

        @pl.loop(0, SBLK)
        def _(i):
            exrow[i, :] = zero16

        plsc.subcore_barrier()

        lanes = jnp.arange(16, dtype=jnp.int32)
        zlanes = jnp.zeros((16,), jnp.int32)

        def body(g_vmem, idx_vmem, ex_vmem):
            for j in range(SBLK // 16):
                plsc.store_scatter(exrow, [j * 16 + lanes, zlanes],
                                   ex_vmem[0, pl.ds(j * 16, 16)])
            pltpu.sync_copy(g_vmem, acc1.at[idx_vmem.at[0]], add=True)
            pltpu.sync_copy(exrow, acc2.at[idx_vmem.at[0]], add=True)

        pltpu.emit_pipeline(
            body,
            grid=(N // SBLK,),
            in_specs=[
                pl.BlockSpec((SBLK, H), index_map=lambda i: (i, 0)),
                pl.BlockSpec((1, SBLK), index_map=lambda i: (i, 0)),
                pl.BlockSpec((1, SBLK), index_map=lambda i: (i, 0)),
            ],
            out_specs=[],
            core_axis_name=("c", "s"),
            dimension_semantics=(pltpu.PARALLEL,),
        )(g_hbm, idx_hbm, ex_hbm)

        plsc.subcore_barrier()
        pltpu.sync_copy(acc1.at[pl.ds(sid * 624, 624)],
                        out1_hbm.at[cid, pl.ds(sid * 624, 624)])
        pltpu.sync_copy(acc2.at[pl.ds(sid * 624, 624)],
                        out2_hbm.at[cid, pl.ds(sid * 624, 624)])

        @pl.when(sid == 15)
        def _():
            pltpu.sync_copy(acc1.at[pl.ds(9984, 16)],
                            out1_hbm.at[cid, pl.ds(9984, 16)])
            pltpu.sync_copy(acc2.at[pl.ds(9984, 16)],
                            out2_hbm.at[cid, pl.ds(9984, 16)])

    return k(g, idx2d, ex2d)


def _rdenom_body(ud_ref, rd_ref):
    d = ud_ref[0, :, 0:1] + ud_ref[1, :, 0:1]
    rd_ref[...] = jnp.where(d > 0.0, 1.0 / d, 0.0)


def _rdenom(u2d):
    return pl.pallas_call(
        _rdenom_body,
        grid=(1,),
        in_specs=[pl.BlockSpec((2, T, DW), lambda i: (0, 0, 0))],
        out_specs=pl.BlockSpec((T, 1), lambda i: (0, 0)),
        out_shape=jax.ShapeDtypeStruct((T, 1), jnp.float32),
    )(u2d)


def _project_body(uh_ref, rd_ref, w3_ref, b3_ref, th_ref):
    uh = (uh_ref[0] + uh_ref[1]) * rd_ref[...]
    th = jax.lax.dot_general(uh.astype(jnp.bfloat16),
                             w3_ref[...].astype(jnp.bfloat16),
                             (((1,), (0,)), ((), ())),
                             preferred_element_type=jnp.float32)
    th_ref[...] = jnp.maximum(th + b3_ref[...], 0.0)


def _project(u2h, rd2d, w3, b3r):
    return pl.pallas_call(
        _project_body,
        grid=(T // TBLK,),
        in_specs=[
            pl.BlockSpec((2, TBLK, H), lambda i: (0, i, 0)),
            pl.BlockSpec((TBLK, 1), lambda i: (i, 0)),
            pl.BlockSpec((H, H), lambda i: (0, 0)),
            pl.BlockSpec((1, H), lambda i: (0, 0)),
        ],
        out_specs=pl.BlockSpec((TBLK, H), lambda i: (i, 0)),
        out_shape=jax.ShapeDtypeStruct((T, H), jnp.float32),
    )(u2h, rd2d, w3, b3r)


def _attn_gather(ex2d, idx2d, rdenom):
    mesh = plsc.VectorSubcoreMesh(core_axis_name="c", subcore_axis_name="s")

    @functools.partial(
        pl.kernel,
        out_type=jax.ShapeDtypeStruct((N // SBLK, SBLK), jnp.float32),
        mesh=mesh,
        scratch_types=[pltpu.VMEM((T,), jnp.float32)],
        compiler_params=pltpu.CompilerParams(use_tc_tiling_on_sc=False,
                                             needs_layout_passes=False),
    )
    def k(ex_hbm, idx_hbm, rd_hbm, out_hbm, rd_vmem):
        pltpu.sync_copy(rd_hbm, rd_vmem)

        def body(ex_vmem, idx_vmem, attn_vmem):
            for j in range(SBLK // 16):
                idxv = idx_vmem[0, pl.ds(j * 16, 16)]
                rd = plsc.load_gather(rd_vmem, [idxv])
                attn_vmem[0, pl.ds(j * 16, 16)] = (
                    ex_vmem[0, pl.ds(j * 16, 16)] * rd)

        pltpu.emit_pipeline(
            body,
            grid=(N // SBLK,),
            in_specs=[
                pl.BlockSpec((1, SBLK), index_map=lambda i: (i, 0)),
                pl.BlockSpec((1, SBLK), index_map=lambda i: (i, 0)),
            ],
            out_specs=[pl.BlockSpec((1, SBLK), index_map=lambda i: (i, 0))],
            core_axis_name=("c", "s"),
            dimension_semantics=(pltpu.PARALLEL,),
        )(ex_hbm, idx_hbm, out_hbm)

    return k(ex2d, idx2d, rdenom)


def kernel(agent_h, team_idx, n_teams, W1, b1, W2, b2, W3, b3):
    idx2d = team_idx.astype(jnp.int32).reshape(N // SBLK, SBLK)
    scale2d = (jnp.asarray(n_teams, jnp.float32) / 10000.0).reshape(1, 1)
    b1r = b1.reshape(1, H // 2)
    w2r = W2.reshape(1, H // 2)
    b2r = b2.reshape(1, 1)
    b3r = b3.reshape(1, H)

    g, ex3d = _scores_prescale(agent_h, W1, b1r, w2r, W2, b2r, scale2d)
    ex2d = ex3d.reshape(N // SBLK, SBLK)
    u2h, u2d = _segment_accumulate(g, idx2d, ex2d)
    rd2d = _rdenom(u2d)
    team_h = _project(u2h, rd2d, W3, b3r)
    attn2d = _attn_gather(ex2d, idx2d, rd2d.reshape(T))
    return team_h, attn2d.reshape(N)


# speedup vs baseline: 10.9480x; 1.0366x over previous
import functools

import jax
import jax.numpy as jnp
from jax import lax
from jax.experimental import pallas as pl
from jax.experimental.pallas import tpu as pltpu
from jax.experimental.pallas import tpu_sc as plsc

N = 100000
H = 128
T = 10000
DW = 16
BLK1 = 20000
SBLK = 80
TBLK = 2000


def _scores_prescale_body(h_ref, w1_ref, b1_ref, w2_ref, w2c_ref, b2_ref,
                          sc_ref, g_ref, ex_ref):
    h = h_ref[...]
    t1 = jnp.tanh(
        jax.lax.dot_general(h.astype(jnp.bfloat16),
                            w1_ref[...].astype(jnp.bfloat16),
                            (((1,), (0,)), ((), ())),
                            preferred_element_type=jnp.float32)
        + b1_ref[...])
    w2 = w2_ref[...]
    scale = sc_ref[0, 0]
    m = (jnp.sum(jnp.abs(w2)) + jnp.abs(b2_ref[0, 0])) * scale
    t1b = t1.astype(jnp.bfloat16)
    w2b = w2.astype(jnp.bfloat16)
    s_col = (jax.lax.dot_general(t1b, w2c_ref[...].astype(jnp.bfloat16),
                                 (((1,), (0,)), ((), ())),
                                 preferred_element_type=jnp.float32)
             + b2_ref[...]) * scale
    ex_col = jnp.exp(s_col - m)
    s_row = (jax.lax.dot_general(w2b, t1b, (((1,), (1,)), ((), ())),
                                 preferred_element_type=jnp.float32)
             + b2_ref[...]) * scale
    ex_row = jnp.exp(s_row - m)
    g_ref[...] = h * ex_col
    ex_ref[0] = ex_row


def _scores_prescale(agent_h, w1, b1r, w2r, w2c, b2r, scale2d):
    return pl.pallas_call(
        _scores_prescale_body,
        grid=(N // BLK1,),
        in_specs=[
            pl.BlockSpec((BLK1, H), lambda i: (i, 0)),
            pl.BlockSpec((H, H // 2), lambda i: (0, 0)),
            pl.BlockSpec((1, H // 2), lambda i: (0, 0)),
            pl.BlockSpec((1, H // 2), lambda i: (0, 0)),
            pl.BlockSpec((H // 2, 1), lambda i: (0, 0)),
            pl.BlockSpec((1, 1), lambda i: (0, 0)),
            pl.BlockSpec((1, 1), lambda i: (0, 0)),
        ],
        out_specs=[
            pl.BlockSpec((BLK1, H), lambda i: (i, 0)),
            pl.BlockSpec((1, 1, BLK1), lambda i: (i, 0, 0)),
        ],
        out_shape=[
            jax.ShapeDtypeStruct((N, H), jnp.float32),
            jax.ShapeDtypeStruct((N // BLK1, 1, BLK1), jnp.float32),
        ],
    )(agent_h, w1, b1r, w2r, w2c, b2r, scale2d)


def _segment_accumulate(g, idx2d, ex2d):
    mesh = plsc.VectorSubcoreMesh(core_axis_name="c", subcore_axis_name="s")

    @functools.partial(
        pl.kernel,
        out_type=[
            jax.ShapeDtypeStruct((2, T, H), jnp.float32),
            jax.ShapeDtypeStruct((2, T, DW), jnp.float32),
        ],
        mesh=mesh,
        scratch_types=[
            pltpu.VMEM_SHARED((T, H), jnp.float32),
            pltpu.VMEM_SHARED((T, DW), jnp.float32),
            pltpu.VMEM((SBLK, DW), jnp.float32),
            pltpu.VMEM((78, H), jnp.float32),
            pltpu.VMEM((78, DW), jnp.float32),
        ],
        compiler_params=pltpu.CompilerParams(use_tc_tiling_on_sc=False,
                                             needs_layout_passes=False),
    )
    def k(g_hbm, idx_hbm, ex_hbm, out1_hbm, out2_hbm,
          acc1, acc2, exrow, zbuf, zbuf2):
        cid = lax.axis_index("c")
        sid = lax.axis_index("s")
        zero16 = jnp.zeros((16,), jnp.float32)

        @pl.loop(0, 78)
        def _(i):
            for c in range(H // 16):
                zbuf[i, pl.ds(c * 16, 16)] = zero16
            zbuf2[i, :] = zero16

        @pl.loop(0, 8)
        def _(kk):
            pltpu.sync_copy(zbuf, acc1.at[pl.ds(sid * 624 + kk * 78, 78)])
            pltpu.sync_copy(zbuf2, acc2.at[pl.ds(sid * 624 + kk * 78, 78)])

        @pl.when(sid == 15)
        def _():
            pltpu.sync_copy(zbuf.at[pl.ds(0, 16)], acc1.at[pl.ds(9984, 16)])
            pltpu.sync_copy(zbuf2.at[pl.ds(0, 16)], acc2.at[pl.ds(9984, 16)])

        @pl.loop(0, SBLK)
        def _(i):
            exrow[i, :] = zero16

        plsc.subcore_barrier()

        lanes = jnp.arange(16, dtype=jnp.int32)
        zlanes = jnp.zeros((16,), jnp.int32)

        def body(g_vmem, idx_vmem, ex_vmem):
            for j in range(SBLK // 16):
                plsc.store_scatter(exrow, [j * 16 + lanes, zlanes],
                                   ex_vmem[0, pl.ds(j * 16, 16)])
            pltpu.sync_copy(g_vmem, acc1.at[idx_vmem.at[0]], add=True)
            pltpu.sync_copy(exrow, acc2.at[idx_vmem.at[0]], add=True)

        pltpu.emit_pipeline(
            body,
            grid=(N // SBLK,),
            in_specs=[
                pl.BlockSpec((SBLK, H), index_map=lambda i: (i, 0)),
                pl.BlockSpec((1, SBLK), index_map=lambda i: (i, 0)),
                pl.BlockSpec((1, SBLK), index_map=lambda i: (i, 0)),
            ],
            out_specs=[],
            core_axis_name=("c", "s"),
            dimension_semantics=(pltpu.PARALLEL,),
        )(g_hbm, idx_hbm, ex_hbm)

        plsc.subcore_barrier()
        pltpu.sync_copy(acc1.at[pl.ds(sid * 624, 624)],
                        out1_hbm.at[cid, pl.ds(sid * 624, 624)])
        pltpu.sync_copy(acc2.at[pl.ds(sid * 624, 624)],
                        out2_hbm.at[cid, pl.ds(sid * 624, 624)])

        @pl.when(sid == 15)
        def _():
            pltpu.sync_copy(acc1.at[pl.ds(9984, 16)],
                            out1_hbm.at[cid, pl.ds(9984, 16)])
            pltpu.sync_copy(acc2.at[pl.ds(9984, 16)],
                            out2_hbm.at[cid, pl.ds(9984, 16)])

    return k(g, idx2d, ex2d)


def _rdenom_body(ud_ref, rd_ref):
    d = ud_ref[0, :, 0:1] + ud_ref[1, :, 0:1]
    rd_ref[...] = jnp.where(d > 0.0, 1.0 / d, 0.0)


def _rdenom(u2d):
    return pl.pallas_call(
        _rdenom_body,
        grid=(1,),
        in_specs=[pl.BlockSpec((2, T, DW), lambda i: (0, 0, 0))],
        out_specs=pl.BlockSpec((T, 1), lambda i: (0, 0)),
        out_shape=jax.ShapeDtypeStruct((T, 1), jnp.float32),
    )(u2d)


def _project_body(uh_ref, rd_ref, w3_ref, b3_ref, th_ref):
    uh = (uh_ref[0] + uh_ref[1]) * rd_ref[...]
    th = jax.lax.dot_general(uh.astype(jnp.bfloat16),
                             w3_ref[...].astype(jnp.bfloat16),
                             (((1,), (0,)), ((), ())),
                             preferred_element_type=jnp.float32)
    th_ref[...] = jnp.maximum(th + b3_ref[...], 0.0)


def _project(u2h, rd2d, w3, b3r):
    return pl.pallas_call(
        _project_body,
        grid=(T // TBLK,),
        in_specs=[
            pl.BlockSpec((2, TBLK, H), lambda i: (0, i, 0)),
            pl.BlockSpec((TBLK, 1), lambda i: (i, 0)),
            pl.BlockSpec((H, H), lambda i: (0, 0)),
            pl.BlockSpec((1, H), lambda i: (0, 0)),
        ],
        out_specs=pl.BlockSpec((TBLK, H), lambda i: (i, 0)),
        out_shape=jax.ShapeDtypeStruct((T, H), jnp.float32),
    )(u2h, rd2d, w3, b3r)


def _attn_gather(ex2d, idx2d, rdenom):
    mesh = plsc.VectorSubcoreMesh(core_axis_name="c", subcore_axis_name="s")

    @functools.partial(
        pl.kernel,
        out_type=jax.ShapeDtypeStruct((N // SBLK, SBLK), jnp.float32),
        mesh=mesh,
        scratch_types=[pltpu.VMEM((T,), jnp.float32)],
        compiler_params=pltpu.CompilerParams(use_tc_tiling_on_sc=False,
                                             needs_layout_passes=False),
    )
    def k(ex_hbm, idx_hbm, rd_hbm, out_hbm, rd_vmem):
        pltpu.sync_copy(rd_hbm, rd_vmem)

        def body(ex_vmem, idx_vmem, attn_vmem):
            for j in range(SBLK // 16):
                idxv = idx_vmem[0, pl.ds(j * 16, 16)]
                rd = plsc.load_gather(rd_vmem, [idxv])
                attn_vmem[0, pl.ds(j * 16, 16)] = (
                    ex_vmem[0, pl.ds(j * 16, 16)] * rd)

        pltpu.emit_pipeline(
            body,
            grid=(N // SBLK,),
            in_specs=[
                pl.BlockSpec((1, SBLK), index_map=lambda i: (i, 0)),
                pl.BlockSpec((1, SBLK), index_map=lambda i: (i, 0)),
            ],
            out_specs=[pl.BlockSpec((1, SBLK), index_map=lambda i: (i, 0))],
            core_axis_name=("c", "s"),
            dimension_semantics=(pltpu.PARALLEL,),
        )(ex_hbm, idx_hbm, out_hbm)

    return k(ex2d, idx2d, rdenom)


def kernel(agent_h, team_idx, n_teams, W1, b1, W2, b2, W3, b3):
    idx2d = team_idx.astype(jnp.int32).reshape(N // SBLK, SBLK)
    scale2d = (jnp.asarray(n_teams, jnp.float32) / 10000.0).reshape(1, 1)
    b1r = b1.reshape(1, H // 2)
    w2r = W2.reshape(1, H // 2)
    b2r = b2.reshape(1, 1)
    b3r = b3.reshape(1, H)

    g, ex3d = _scores_prescale(agent_h, W1, b1r, w2r, W2, b2r, scale2d)
    ex2d = ex3d.reshape(N // SBLK, SBLK)
    u2h, u2d = _segment_accumulate(g, idx2d, ex2d)
    rd2d = _rdenom(u2d)
    team_h = _project(u2h, rd2d, W3, b3r)
    attn2d = _attn_gather(ex2d, idx2d, rd2d.reshape(T))
    return team_h, attn2d.reshape(N)
